# baseline (speedup 1.0000x reference)
import functools

import jax
import jax.numpy as jnp
from jax import lax
from jax.experimental import pallas as pl
from jax.experimental.pallas import tpu as pltpu
from jax.experimental.pallas import tpu_sc as plsc

N = 10000
D = 256
H = 5
E = 160000

HP = 8
NPAD = 10240
NS = 16
CHUNK = 128
EC = 10240
NCH = EC // CHUNK
EPAD = EC * NS
PADIDX = NPAD - 8
RPT = NPAD // NS
NV = RPT * HP // 16


def _rsqrt16(d):
    i = plsc.bitcast(d, jnp.int32)
    y = plsc.bitcast(0x5F3759DF - (i >> 1), jnp.float32)
    for _ in range(3):
        y = y * (1.5 - 0.5 * d * y * y)
    return y


def _fused_sc_body(h1_hbm, src_hbm, dst_hbm, ones_hbm, zeros_hbm,
                   b1_hbm, p2_hbm, g1_hbm, g2_hbm,
                   idx_s_v, idx_d_v, rows_v, gbuf, dinvbuf, abuf, zbuf,
                   ones_v, dummy_v, b1buf, acc_sh, gsem, ssem):
    sid = lax.axis_index("s")
    sl = pl.ds(sid * RPT, RPT)

    lane = lax.iota(jnp.int32, 16)
    cols = lane & 7
    rowpat = lane >> 3

    pltpu.sync_copy(src_hbm.at[sid], idx_s_v)
    pltpu.sync_copy(dst_hbm.at[sid], idx_d_v)
    pltpu.sync_copy(ones_hbm, ones_v)
    pltpu.sync_copy(zeros_hbm, zbuf)
    pltpu.sync_copy(b1_hbm, b1buf)
    pltpu.sync_copy(zbuf, acc_sh.at[sl])
    plsc.subcore_barrier()

    def _drain_scatters(n, desc_src, desc_dst):
        def d(j, c):
            pltpu.make_async_copy(desc_src, desc_dst, ssem).wait()
            return c
        lax.fori_loop(0, n, d, 0)

    def _propagate(tab_hbm):
        def fire(j, c):
            pltpu.async_copy(tab_hbm.at[idx_s_v.at[j]], rows_v.at[j], gsem)
            return c
        lax.fori_loop(0, NCH, fire, 0)

        def scat(j, c):
            pltpu.make_async_copy(
                tab_hbm.at[pl.ds(0, CHUNK)], rows_v.at[j], gsem).wait()
            pltpu.async_copy(
                rows_v.at[j], acc_sh.at[idx_d_v.at[j]], ssem, add=True)
            return c
        lax.fori_loop(0, NCH, scat, 0)
        _drain_scatters(NCH, zeros_hbm.at[pl.ds(0, CHUNK)], dummy_v)

    scope = jax.named_scope
    def deg_scat(j, c):
        pltpu.async_copy(ones_v, acc_sh.at[idx_d_v.at[j]], ssem, add=True)
        return c
    with scope("ph_deg"):
        lax.fori_loop(0, NCH, deg_scat, 0)
        _drain_scatters(NCH, ones_hbm, ones_v)
    plsc.subcore_barrier()

    pltpu.sync_copy(acc_sh.at[sl], abuf)
    pltpu.sync_copy(zbuf, acc_sh.at[sl])
    pltpu.sync_copy(h1_hbm.at[sl], dinvbuf)

    def ew_a(i, c):
        rows = rowpat + 2 * i
        d = plsc.load_gather(abuf, [rows, cols]) + 1.0
        h = plsc.load_gather(dinvbuf, [rows, cols])
        y = _rsqrt16(d)
        plsc.store_scatter(dinvbuf, [rows, cols], y)
        plsc.store_scatter(gbuf, [rows, cols], y * h)
        return c
    with scope("ph_ew_a"):
        lax.fori_loop(0, NV, ew_a, 0)
    pltpu.sync_copy(gbuf, g1_hbm.at[sl])
    plsc.subcore_barrier()

    with scope("ph_pass1"):
        _propagate(g1_hbm)
    plsc.subcore_barrier()

    pltpu.sync_copy(acc_sh.at[sl], abuf)
    pltpu.sync_copy(zbuf, acc_sh.at[sl])
    b1v = b1buf[...]

    def ew_b(i, c):
        rows = rowpat + 2 * i
        a = plsc.load_gather(abuf, [rows, cols])
        g = plsc.load_gather(gbuf, [rows, cols])
        y = plsc.load_gather(dinvbuf, [rows, cols])
        p = y * (a + g) + b1v
        plsc.store_scatter(gbuf, [rows, cols], y * jnp.maximum(p, 0.0))
        return c
    with scope("ph_ew_b"):
        lax.fori_loop(0, NV, ew_b, 0)
    pltpu.sync_copy(gbuf, g2_hbm.at[sl])
    plsc.subcore_barrier()

    with scope("ph_pass2"):
        _propagate(g2_hbm)
    plsc.subcore_barrier()

    pltpu.sync_copy(acc_sh.at[sl], abuf)

    def ew_c(i, c):
        rows = rowpat + 2 * i
        a = plsc.load_gather(abuf, [rows, cols])
        g = plsc.load_gather(gbuf, [rows, cols])
        y = plsc.load_gather(dinvbuf, [rows, cols])
        plsc.store_scatter(abuf, [rows, cols], y * (a + g))
        return c
    lax.fori_loop(0, NV, ew_c, 0)
    pltpu.sync_copy(abuf, p2_hbm.at[sl])


_fused_sc = pl.kernel(
    _fused_sc_body,
    out_type=(jax.ShapeDtypeStruct((NPAD, HP), jnp.float32),
              jax.ShapeDtypeStruct((NPAD, HP), jnp.float32),
              jax.ShapeDtypeStruct((NPAD, HP), jnp.float32)),
    name="gcn_fused_sc",
    mesh=plsc.VectorSubcoreMesh(
        core_axis_name="c", subcore_axis_name="s",
        num_cores=1, num_subcores=NS),
    scratch_types=[
        pltpu.VMEM((NCH, CHUNK), jnp.int32),
        pltpu.VMEM((NCH, CHUNK), jnp.int32),
        pltpu.VMEM((NCH, CHUNK, HP), jnp.float32),
        pltpu.VMEM((RPT, HP), jnp.float32),
        pltpu.VMEM((RPT, HP), jnp.float32),
        pltpu.VMEM((RPT, HP), jnp.float32),
        pltpu.VMEM((RPT, HP), jnp.float32),
        pltpu.VMEM((CHUNK, HP), jnp.float32),
        pltpu.VMEM((CHUNK, HP), jnp.float32),
        pltpu.VMEM((16,), jnp.float32),
        pltpu.VMEM_SHARED((NPAD, HP), jnp.float32),
        pltpu.SemaphoreType.DMA,
        pltpu.SemaphoreType.DMA,
    ],
    compiler_params=pltpu.CompilerParams(
        use_tc_tiling_on_sc=False, needs_layout_passes=False),
)


def _mm1_body(x_ref, w_ref, o_ref):
    o_ref[...] = jnp.dot(x_ref[...], w_ref[...],
                         preferred_element_type=jnp.float32)


def _final_body(p2_ref, w2_ref, b2_ref, o_ref):
    o_ref[...] = jnp.dot(p2_ref[...], w2_ref[...],
                         preferred_element_type=jnp.float32) + b2_ref[...]


_RB = 1000
_MB = 640


def kernel(t, x, edge_index, W1, b1, W2, b2):
    del t
    f32 = jnp.float32

    src = edge_index[0].astype(jnp.int32)
    dst = edge_index[1].astype(jnp.int32)
    padv = PADIDX + (jnp.arange(EPAD - E, dtype=jnp.int32) % 8)
    src_t = jnp.concatenate([src, padv]).reshape(NS, NCH, CHUNK)
    dst_t = jnp.concatenate([dst, padv]).reshape(NS, NCH, CHUNK)

    W1p = jnp.zeros((D, HP), f32).at[:, :H].set(W1)
    W2p = jnp.zeros((HP, D), f32).at[:H, :].set(W2)
    b1v = jnp.zeros((16,), f32).at[:H].set(b1).at[8:8 + H].set(b1)
    b2r = b2.reshape(1, D)

    ones_c = jnp.ones((CHUNK, HP), f32)
    zeros_r = jnp.zeros((RPT, HP), f32)

    h1p = pl.pallas_call(
        _mm1_body,
        grid=(NPAD // _MB,),
        in_specs=[pl.BlockSpec((_MB, D), lambda i: (i, 0)),
                  pl.BlockSpec((D, HP), lambda i: (0, 0))],
        out_specs=pl.BlockSpec((_MB, HP), lambda i: (i, 0)),
        out_shape=jax.ShapeDtypeStruct((NPAD, HP), f32),
    )(x, W1p)

    p2, _, _ = _fused_sc(h1p, src_t, dst_t, ones_c, zeros_r, b1v)

    out = pl.pallas_call(
        _final_body,
        grid=(N // _RB,),
        in_specs=[pl.BlockSpec((_RB, HP), lambda i: (i, 0)),
                  pl.BlockSpec((HP, D), lambda i: (0, 0)),
                  pl.BlockSpec((1, D), lambda i: (0, 0))],
        out_specs=pl.BlockSpec((_RB, D), lambda i: (i, 0)),
        out_shape=jax.ShapeDtypeStruct((N, D), f32),
    )(p2, W2p, b2r)
    return out


# speedup vs baseline: 1.0099x; 1.0099x over previous
import functools

import jax
import jax.numpy as jnp
from jax import lax
from jax.experimental import pallas as pl
from jax.experimental.pallas import tpu as pltpu
from jax.experimental.pallas import tpu_sc as plsc

N = 10000
D = 256
H = 5
E = 160000

HP = 8
NPAD = 10240
NS = 16
CHUNK = 128
EC = 10240
NCH = EC // CHUNK
EPAD = EC * NS
PADIDX = NPAD - 8
RPT = NPAD // NS
NV = RPT * HP // 16


def _rsqrt16(d):
    i = plsc.bitcast(d, jnp.int32)
    y = plsc.bitcast(0x5F3759DF - (i >> 1), jnp.float32)
    for _ in range(2):
        y = y * (1.5 - 0.5 * d * y * y)
    return y


def _fused_sc_body(h1_hbm, src_hbm, dst_hbm, ones_hbm, zeros_hbm,
                   b1_hbm, p2_hbm, g1_hbm, g2_hbm,
                   idx_s_v, idx_d_v, rows_v, gbuf, dinvbuf, abuf, zbuf,
                   ones_v, dummy_v, b1buf, acc_sh, gsem, ssem):
    sid = lax.axis_index("s")
    sl = pl.ds(sid * RPT, RPT)

    lane = lax.iota(jnp.int32, 16)
    cols = lane & 7
    rowpat = lane >> 3

    pltpu.sync_copy(src_hbm.at[sid], idx_s_v)
    pltpu.sync_copy(dst_hbm.at[sid], idx_d_v)
    pltpu.sync_copy(ones_hbm, ones_v)
    pltpu.sync_copy(zeros_hbm, zbuf)
    pltpu.sync_copy(b1_hbm, b1buf)
    pltpu.sync_copy(zbuf, acc_sh.at[sl])
    plsc.subcore_barrier()

    def _drain_scatters(n, desc_src, desc_dst):
        def d(j, c):
            pltpu.make_async_copy(desc_src, desc_dst, ssem).wait()
            return c
        lax.fori_loop(0, n, d, 0)

    def _propagate(tab_hbm):
        def fire(j, c):
            pltpu.async_copy(tab_hbm.at[idx_s_v.at[j]], rows_v.at[j], gsem)
            return c
        lax.fori_loop(0, NCH, fire, 0)

        def scat(j, c):
            pltpu.make_async_copy(
                tab_hbm.at[pl.ds(0, CHUNK)], rows_v.at[j], gsem).wait()
            pltpu.async_copy(
                rows_v.at[j], acc_sh.at[idx_d_v.at[j]], ssem, add=True)
            return c
        lax.fori_loop(0, NCH, scat, 0)
        _drain_scatters(NCH, zeros_hbm.at[pl.ds(0, CHUNK)], dummy_v)

    scope = jax.named_scope
    def deg_scat(j, c):
        pltpu.async_copy(ones_v, acc_sh.at[idx_d_v.at[j]], ssem, add=True)
        return c
    with scope("ph_deg"):
        lax.fori_loop(0, NCH, deg_scat, 0)
        _drain_scatters(NCH, ones_hbm, ones_v)
    plsc.subcore_barrier()

    pltpu.sync_copy(acc_sh.at[sl], abuf)
    pltpu.sync_copy(zbuf, acc_sh.at[sl])
    pltpu.sync_copy(h1_hbm.at[sl], dinvbuf)

    def ew_a(i, c):
        rows = rowpat + 2 * i
        d = plsc.load_gather(abuf, [rows, cols]) + 1.0
        h = plsc.load_gather(dinvbuf, [rows, cols])
        y = _rsqrt16(d)
        plsc.store_scatter(dinvbuf, [rows, cols], y)
        plsc.store_scatter(gbuf, [rows, cols], y * h)
        return c
    with scope("ph_ew_a"):
        lax.fori_loop(0, NV, ew_a, 0, unroll=4)
    pltpu.sync_copy(gbuf, g1_hbm.at[sl])
    plsc.subcore_barrier()

    with scope("ph_pass1"):
        _propagate(g1_hbm)
    plsc.subcore_barrier()

    pltpu.sync_copy(acc_sh.at[sl], abuf)
    pltpu.sync_copy(zbuf, acc_sh.at[sl])
    b1v = b1buf[...]

    def ew_b(i, c):
        rows = rowpat + 2 * i
        a = plsc.load_gather(abuf, [rows, cols])
        g = plsc.load_gather(gbuf, [rows, cols])
        y = plsc.load_gather(dinvbuf, [rows, cols])
        p = y * (a + g) + b1v
        plsc.store_scatter(gbuf, [rows, cols], y * jnp.maximum(p, 0.0))
        return c
    with scope("ph_ew_b"):
        lax.fori_loop(0, NV, ew_b, 0, unroll=4)
    pltpu.sync_copy(gbuf, g2_hbm.at[sl])
    plsc.subcore_barrier()

    with scope("ph_pass2"):
        _propagate(g2_hbm)
    plsc.subcore_barrier()

    pltpu.sync_copy(acc_sh.at[sl], abuf)

    def ew_c(i, c):
        rows = rowpat + 2 * i
        a = plsc.load_gather(abuf, [rows, cols])
        g = plsc.load_gather(gbuf, [rows, cols])
        y = plsc.load_gather(dinvbuf, [rows, cols])
        plsc.store_scatter(abuf, [rows, cols], y * (a + g))
        return c
    lax.fori_loop(0, NV, ew_c, 0, unroll=4)
    pltpu.sync_copy(abuf, p2_hbm.at[sl])


_fused_sc = pl.kernel(
    _fused_sc_body,
    out_type=(jax.ShapeDtypeStruct((NPAD, HP), jnp.float32),
              jax.ShapeDtypeStruct((NPAD, HP), jnp.float32),
              jax.ShapeDtypeStruct((NPAD, HP), jnp.float32)),
    name="gcn_fused_sc",
    mesh=plsc.VectorSubcoreMesh(
        core_axis_name="c", subcore_axis_name="s",
        num_cores=1, num_subcores=NS),
    scratch_types=[
        pltpu.VMEM((NCH, CHUNK), jnp.int32),
        pltpu.VMEM((NCH, CHUNK), jnp.int32),
        pltpu.VMEM((NCH, CHUNK, HP), jnp.float32),
        pltpu.VMEM((RPT, HP), jnp.float32),
        pltpu.VMEM((RPT, HP), jnp.float32),
        pltpu.VMEM((RPT, HP), jnp.float32),
        pltpu.VMEM((RPT, HP), jnp.float32),
        pltpu.VMEM((CHUNK, HP), jnp.float32),
        pltpu.VMEM((CHUNK, HP), jnp.float32),
        pltpu.VMEM((16,), jnp.float32),
        pltpu.VMEM_SHARED((NPAD, HP), jnp.float32),
        pltpu.SemaphoreType.DMA,
        pltpu.SemaphoreType.DMA,
    ],
    compiler_params=pltpu.CompilerParams(
        use_tc_tiling_on_sc=False, needs_layout_passes=False),
)


def _mm1_body(x_ref, w_ref, o_ref):
    o_ref[...] = jnp.dot(x_ref[...], w_ref[...],
                         preferred_element_type=jnp.float32)


def _final_body(p2_ref, w2_ref, b2_ref, o_ref):
    o_ref[...] = jnp.dot(p2_ref[...], w2_ref[...],
                         preferred_element_type=jnp.float32) + b2_ref[...]


_RB = 1000
_MB = 640


def kernel(t, x, edge_index, W1, b1, W2, b2):
    del t
    f32 = jnp.float32

    src = edge_index[0].astype(jnp.int32)
    dst = edge_index[1].astype(jnp.int32)
    padv = PADIDX + (jnp.arange(EPAD - E, dtype=jnp.int32) % 8)
    src_t = jnp.concatenate([src, padv]).reshape(NS, NCH, CHUNK)
    dst_t = jnp.concatenate([dst, padv]).reshape(NS, NCH, CHUNK)

    W1p = jnp.zeros((D, HP), f32).at[:, :H].set(W1)
    W2p = jnp.zeros((HP, D), f32).at[:H, :].set(W2)
    b1v = jnp.zeros((16,), f32).at[:H].set(b1).at[8:8 + H].set(b1)
    b2r = b2.reshape(1, D)

    ones_c = jnp.ones((CHUNK, HP), f32)
    zeros_r = jnp.zeros((RPT, HP), f32)

    h1p = pl.pallas_call(
        _mm1_body,
        grid=(NPAD // _MB,),
        in_specs=[pl.BlockSpec((_MB, D), lambda i: (i, 0)),
                  pl.BlockSpec((D, HP), lambda i: (0, 0))],
        out_specs=pl.BlockSpec((_MB, HP), lambda i: (i, 0)),
        out_shape=jax.ShapeDtypeStruct((NPAD, HP), f32),
    )(x, W1p)

    p2, _, _ = _fused_sc(h1p, src_t, dst_t, ones_c, zeros_r, b1v)

    out = pl.pallas_call(
        _final_body,
        grid=(N // _RB,),
        in_specs=[pl.BlockSpec((_RB, HP), lambda i: (i, 0)),
                  pl.BlockSpec((HP, D), lambda i: (0, 0)),
                  pl.BlockSpec((1, D), lambda i: (0, 0))],
        out_specs=pl.BlockSpec((_RB, D), lambda i: (i, 0)),
        out_shape=jax.ShapeDtypeStruct((N, D), f32),
    )(p2, W2p, b2r)
    return out


# speedup vs baseline: 1.3914x; 1.3779x over previous
import functools

import jax
import jax.numpy as jnp
from jax import lax
from jax.experimental import pallas as pl
from jax.experimental.pallas import tpu as pltpu
from jax.experimental.pallas import tpu_sc as plsc

N = 10000
D = 256
H = 5
E = 160000

HP = 8
NPAD = 10240
NS = 16
CHUNK = 128
EC = 10240
NCH = EC // CHUNK
EPAD = EC * NS
PADIDX = NPAD - 8
RPT = NPAD // NS
NV = RPT * HP // 16
RING = 16
LOOK = 8


def _rsqrt16(d):
    i = plsc.bitcast(d, jnp.int32)
    y = plsc.bitcast(0x5F3759DF - (i >> 1), jnp.float32)
    for _ in range(2):
        y = y * (1.5 - 0.5 * d * y * y)
    return y


def _fused_sc_body(h1_hbm, src_hbm, dst_hbm, ones_hbm, zeros_hbm,
                   b1_hbm, p2_hbm,
                   idx_s_v, idx_d_v, rows_v, gbuf, dinvbuf, abuf, zbuf,
                   ones_v, dummy_v, b1buf, acc_sh, g_sh, gsem, ssem):
    sid = lax.axis_index("s")
    sl = pl.ds(sid * RPT, RPT)

    lane = lax.iota(jnp.int32, 16)
    cols = lane & 7
    rowpat = lane >> 3

    pltpu.sync_copy(src_hbm.at[sid], idx_s_v)
    pltpu.sync_copy(dst_hbm.at[sid], idx_d_v)
    pltpu.sync_copy(ones_hbm, ones_v)
    pltpu.sync_copy(zeros_hbm, zbuf)
    pltpu.sync_copy(b1_hbm, b1buf)
    pltpu.sync_copy(zbuf, acc_sh.at[sl])
    plsc.subcore_barrier()

    def _drain_scatters(n, desc_src, desc_dst):
        def d(j, c):
            pltpu.make_async_copy(desc_src, desc_dst, ssem).wait()
            return c
        lax.fori_loop(0, n, d, 0)

    def _propagate(tab_sh):
        gdesc_src = tab_sh.at[pl.ds(0, CHUNK)]
        sdesc_src = zeros_hbm.at[pl.ds(0, CHUNK)]

        def prime(j, c):
            pltpu.async_copy(tab_sh.at[idx_s_v.at[j]], rows_v.at[j], gsem)
            return c
        lax.fori_loop(0, LOOK, prime, 0)

        def step(j, c):
            pltpu.make_async_copy(gdesc_src, rows_v.at[0], gsem).wait()
            pltpu.async_copy(
                rows_v.at[j & (RING - 1)], acc_sh.at[idx_d_v.at[j]],
                ssem, add=True)

            @pl.when(j >= LOOK)
            def _():
                pltpu.make_async_copy(sdesc_src, dummy_v, ssem).wait()

            @pl.when(j + LOOK < NCH)
            def _():
                pltpu.async_copy(
                    tab_sh.at[idx_s_v.at[j + LOOK]],
                    rows_v.at[(j + LOOK) & (RING - 1)], gsem)
            return c
        lax.fori_loop(0, NCH, step, 0)
        _drain_scatters(LOOK, sdesc_src, dummy_v)

    scope = jax.named_scope
    def deg_scat(j, c):
        pltpu.async_copy(ones_v, acc_sh.at[idx_d_v.at[j]], ssem, add=True)
        return c
    with scope("ph_deg"):
        lax.fori_loop(0, NCH, deg_scat, 0)
        _drain_scatters(NCH, ones_hbm, ones_v)
    plsc.subcore_barrier()

    pltpu.sync_copy(acc_sh.at[sl], abuf)
    pltpu.sync_copy(zbuf, acc_sh.at[sl])
    pltpu.sync_copy(h1_hbm.at[sl], dinvbuf)

    def ew_a(i, c):
        rows = rowpat + 2 * i
        d = plsc.load_gather(abuf, [rows, cols]) + 1.0
        h = plsc.load_gather(dinvbuf, [rows, cols])
        y = _rsqrt16(d)
        plsc.store_scatter(dinvbuf, [rows, cols], y)
        plsc.store_scatter(gbuf, [rows, cols], y * h)
        return c
    with scope("ph_ew_a"):
        lax.fori_loop(0, NV, ew_a, 0, unroll=4)
    pltpu.sync_copy(gbuf, g_sh.at[sl])
    plsc.subcore_barrier()

    with scope("ph_pass1"):
        _propagate(g_sh)
    plsc.subcore_barrier()

    pltpu.sync_copy(acc_sh.at[sl], abuf)
    pltpu.sync_copy(zbuf, acc_sh.at[sl])
    b1v = b1buf[...]

    def ew_b(i, c):
        rows = rowpat + 2 * i
        a = plsc.load_gather(abuf, [rows, cols])
        g = plsc.load_gather(gbuf, [rows, cols])
        y = plsc.load_gather(dinvbuf, [rows, cols])
        p = y * (a + g) + b1v
        plsc.store_scatter(gbuf, [rows, cols], y * jnp.maximum(p, 0.0))
        return c
    with scope("ph_ew_b"):
        lax.fori_loop(0, NV, ew_b, 0, unroll=4)
    pltpu.sync_copy(gbuf, g_sh.at[sl])
    plsc.subcore_barrier()

    with scope("ph_pass2"):
        _propagate(g_sh)
    plsc.subcore_barrier()

    pltpu.sync_copy(acc_sh.at[sl], abuf)

    def ew_c(i, c):
        rows = rowpat + 2 * i
        a = plsc.load_gather(abuf, [rows, cols])
        g = plsc.load_gather(gbuf, [rows, cols])
        y = plsc.load_gather(dinvbuf, [rows, cols])
        plsc.store_scatter(abuf, [rows, cols], y * (a + g))
        return c
    lax.fori_loop(0, NV, ew_c, 0, unroll=4)
    pltpu.sync_copy(abuf, p2_hbm.at[sl])


_fused_sc = pl.kernel(
    _fused_sc_body,
    out_type=jax.ShapeDtypeStruct((NPAD, HP), jnp.float32),
    name="gcn_fused_sc",
    mesh=plsc.VectorSubcoreMesh(
        core_axis_name="c", subcore_axis_name="s",
        num_cores=1, num_subcores=NS),
    scratch_types=[
        pltpu.VMEM((NCH, CHUNK), jnp.int32),
        pltpu.VMEM((NCH, CHUNK), jnp.int32),
        pltpu.VMEM((RING, CHUNK, HP), jnp.float32),
        pltpu.VMEM((RPT, HP), jnp.float32),
        pltpu.VMEM((RPT, HP), jnp.float32),
        pltpu.VMEM((RPT, HP), jnp.float32),
        pltpu.VMEM((RPT, HP), jnp.float32),
        pltpu.VMEM((CHUNK, HP), jnp.float32),
        pltpu.VMEM((CHUNK, HP), jnp.float32),
        pltpu.VMEM((16,), jnp.float32),
        pltpu.VMEM_SHARED((NPAD, HP), jnp.float32),
        pltpu.VMEM_SHARED((NPAD, HP), jnp.float32),
        pltpu.SemaphoreType.DMA,
        pltpu.SemaphoreType.DMA,
    ],
    compiler_params=pltpu.CompilerParams(
        use_tc_tiling_on_sc=False, needs_layout_passes=False),
)


def _mm1_body(x_ref, w_ref, o_ref):
    o_ref[...] = jnp.dot(x_ref[...], w_ref[...],
                         preferred_element_type=jnp.float32)


def _final_body(p2_ref, w2_ref, b2_ref, o_ref):
    o_ref[...] = jnp.dot(p2_ref[...], w2_ref[...],
                         preferred_element_type=jnp.float32) + b2_ref[...]


_RB = 1000
_MB = 640


def kernel(t, x, edge_index, W1, b1, W2, b2):
    del t
    f32 = jnp.float32

    src = edge_index[0].astype(jnp.int32)
    dst = edge_index[1].astype(jnp.int32)
    padv = PADIDX + (jnp.arange(EPAD - E, dtype=jnp.int32) % 8)
    src_t = jnp.concatenate([src, padv]).reshape(NS, NCH, CHUNK)
    dst_t = jnp.concatenate([dst, padv]).reshape(NS, NCH, CHUNK)

    W1p = jnp.zeros((D, HP), f32).at[:, :H].set(W1)
    W2p = jnp.zeros((HP, D), f32).at[:H, :].set(W2)
    b1v = jnp.zeros((16,), f32).at[:H].set(b1).at[8:8 + H].set(b1)
    b2r = b2.reshape(1, D)

    ones_c = jnp.ones((CHUNK, HP), f32)
    zeros_r = jnp.zeros((RPT, HP), f32)

    h1p = pl.pallas_call(
        _mm1_body,
        grid=(NPAD // _MB,),
        in_specs=[pl.BlockSpec((_MB, D), lambda i: (i, 0)),
                  pl.BlockSpec((D, HP), lambda i: (0, 0))],
        out_specs=pl.BlockSpec((_MB, HP), lambda i: (i, 0)),
        out_shape=jax.ShapeDtypeStruct((NPAD, HP), f32),
    )(x, W1p)

    p2 = _fused_sc(h1p, src_t, dst_t, ones_c, zeros_r, b1v)

    out = pl.pallas_call(
        _final_body,
        grid=(N // _RB,),
        in_specs=[pl.BlockSpec((_RB, HP), lambda i: (i, 0)),
                  pl.BlockSpec((HP, D), lambda i: (0, 0)),
                  pl.BlockSpec((1, D), lambda i: (0, 0))],
        out_specs=pl.BlockSpec((_RB, D), lambda i: (i, 0)),
        out_shape=jax.ShapeDtypeStruct((N, D), f32),
    )(p2, W2p, b2r)
    return out


# speedup vs baseline: 1.5188x; 1.0916x over previous
import functools

import jax
import jax.numpy as jnp
from jax import lax
from jax.experimental import pallas as pl
from jax.experimental.pallas import tpu as pltpu
from jax.experimental.pallas import tpu_sc as plsc

N = 10000
D = 256
H = 5
E = 160000

HP = 8
NPAD = 10240
NS = 16
CHUNK = 128
EC = 10240
NCH = EC // CHUNK
EPAD = EC * NS
PADIDX = NPAD - 8
RPT = NPAD // NS
NV = RPT * HP // 16
RING = 16
LOOK = 8


def _rsqrt16(d):
    i = plsc.bitcast(d, jnp.int32)
    y = plsc.bitcast(0x5F3759DF - (i >> 1), jnp.float32)
    for _ in range(2):
        y = y * (1.5 - 0.5 * d * y * y)
    return y


def _fused_sc_body(h1_hbm, src_hbm, dst_hbm, ones_hbm, zeros_hbm,
                   b1_hbm, p2_hbm,
                   idx_s_v, idx_d_v, rows_v, gbuf, dinvbuf, abuf, zbuf,
                   ones_v, dummy_v, b1buf, acc_sh, g_sh, gsem, ssem):
    sid = lax.axis_index("s")
    sl = pl.ds(sid * RPT, RPT)

    lane = lax.iota(jnp.int32, 16)
    cols = lane & 7
    rowpat = lane >> 3

    pltpu.sync_copy(src_hbm.at[sid], idx_s_v)
    pltpu.sync_copy(dst_hbm.at[sid], idx_d_v)
    pltpu.sync_copy(ones_hbm, ones_v)
    pltpu.sync_copy(zeros_hbm, zbuf)
    pltpu.sync_copy(b1_hbm, b1buf)
    pltpu.sync_copy(zbuf, acc_sh.at[sl])
    plsc.subcore_barrier()

    def _drain_scatters(n, desc_src, desc_dst):
        def d(j, c):
            pltpu.make_async_copy(desc_src, desc_dst, ssem).wait()
            return c
        lax.fori_loop(0, n, d, 0)

    def _propagate(tab_sh):
        gdesc_src = tab_sh.at[pl.ds(0, CHUNK)]
        sdesc_src = zeros_hbm.at[pl.ds(0, CHUNK)]

        def prime(j, c):
            pltpu.async_copy(tab_sh.at[idx_s_v.at[j]], rows_v.at[j], gsem)
            return c
        lax.fori_loop(0, LOOK, prime, 0)

        def step(j, c):
            pltpu.make_async_copy(gdesc_src, rows_v.at[0], gsem).wait()
            pltpu.async_copy(
                rows_v.at[j & (RING - 1)], acc_sh.at[idx_d_v.at[j]],
                ssem, add=True)

            @pl.when(j >= LOOK)
            def _():
                pltpu.make_async_copy(sdesc_src, dummy_v, ssem).wait()

            @pl.when(j + LOOK < NCH)
            def _():
                pltpu.async_copy(
                    tab_sh.at[idx_s_v.at[j + LOOK]],
                    rows_v.at[(j + LOOK) & (RING - 1)], gsem)
            return c
        lax.fori_loop(0, NCH, step, 0)
        _drain_scatters(LOOK, sdesc_src, dummy_v)

    scope = jax.named_scope
    def deg_scat(j, c):
        pltpu.async_copy(ones_v, acc_sh.at[idx_d_v.at[j]], ssem, add=True)
        return c
    with scope("ph_deg"):
        lax.fori_loop(0, NCH, deg_scat, 0)
        _drain_scatters(NCH, ones_hbm, ones_v)
    plsc.subcore_barrier()

    pltpu.sync_copy(acc_sh.at[sl], abuf)
    pltpu.sync_copy(zbuf, acc_sh.at[sl])
    pltpu.sync_copy(h1_hbm.at[sl], dinvbuf)

    def ew_a(i, c):
        rows = rowpat + 2 * i
        d = plsc.load_gather(abuf, [rows, cols]) + 1.0
        h = plsc.load_gather(dinvbuf, [rows, cols])
        y = _rsqrt16(d)
        plsc.store_scatter(dinvbuf, [rows, cols], y)
        plsc.store_scatter(gbuf, [rows, cols], y * h)
        return c
    with scope("ph_ew_a"):
        lax.fori_loop(0, NV, ew_a, 0, unroll=4)
    pltpu.sync_copy(gbuf, g_sh.at[sl])
    plsc.subcore_barrier()

    with scope("ph_pass1"):
        _propagate(g_sh)
    plsc.subcore_barrier()

    pltpu.sync_copy(acc_sh.at[sl], abuf)
    pltpu.sync_copy(zbuf, acc_sh.at[sl])
    b1v = b1buf[...]

    def ew_b(i, c):
        rows = rowpat + 2 * i
        a = plsc.load_gather(abuf, [rows, cols])
        g = plsc.load_gather(gbuf, [rows, cols])
        y = plsc.load_gather(dinvbuf, [rows, cols])
        p = y * (a + g) + b1v
        plsc.store_scatter(gbuf, [rows, cols], y * jnp.maximum(p, 0.0))
        return c
    with scope("ph_ew_b"):
        lax.fori_loop(0, NV, ew_b, 0, unroll=4)
    pltpu.sync_copy(gbuf, g_sh.at[sl])
    plsc.subcore_barrier()

    with scope("ph_pass2"):
        _propagate(g_sh)
    plsc.subcore_barrier()

    pltpu.sync_copy(acc_sh.at[sl], abuf)

    def ew_c(i, c):
        rows = rowpat + 2 * i
        a = plsc.load_gather(abuf, [rows, cols])
        g = plsc.load_gather(gbuf, [rows, cols])
        y = plsc.load_gather(dinvbuf, [rows, cols])
        plsc.store_scatter(abuf, [rows, cols], y * (a + g))
        return c
    lax.fori_loop(0, NV, ew_c, 0, unroll=4)
    pltpu.sync_copy(abuf, p2_hbm.at[sl])


_fused_sc = pl.kernel(
    _fused_sc_body,
    out_type=jax.ShapeDtypeStruct((NPAD, HP), jnp.float32),
    name="gcn_fused_sc",
    mesh=plsc.VectorSubcoreMesh(
        core_axis_name="c", subcore_axis_name="s",
        num_cores=1, num_subcores=NS),
    scratch_types=[
        pltpu.VMEM((NCH, CHUNK), jnp.int32),
        pltpu.VMEM((NCH, CHUNK), jnp.int32),
        pltpu.VMEM((RING, CHUNK, HP), jnp.float32),
        pltpu.VMEM((RPT, HP), jnp.float32),
        pltpu.VMEM((RPT, HP), jnp.float32),
        pltpu.VMEM((RPT, HP), jnp.float32),
        pltpu.VMEM((RPT, HP), jnp.float32),
        pltpu.VMEM((CHUNK, HP), jnp.float32),
        pltpu.VMEM((CHUNK, HP), jnp.float32),
        pltpu.VMEM((16,), jnp.float32),
        pltpu.VMEM_SHARED((NPAD, HP), jnp.float32),
        pltpu.VMEM_SHARED((NPAD, HP), jnp.float32),
        pltpu.SemaphoreType.DMA,
        pltpu.SemaphoreType.DMA,
    ],
    compiler_params=pltpu.CompilerParams(
        use_tc_tiling_on_sc=False, needs_layout_passes=False),
)


def _mm1_body(x_ref, w_ref, e_ref, o_ref, src_ref, dst_ref):
    o_ref[...] = jnp.dot(x_ref[...], w_ref[...],
                         preferred_element_type=jnp.float32)
    flat = (pl.program_id(0) * EC
            + lax.broadcasted_iota(jnp.int32, (1, 1, EC), 2))
    pad = PADIDX + (flat & 7)
    real = flat < E
    src_ref[...] = jnp.where(real, e_ref[0:1, :][None], pad)
    dst_ref[...] = jnp.where(real, e_ref[1:2, :][None], pad)


def _final_body(p2_ref, w2_ref, b2_ref, o_ref):
    o_ref[...] = jnp.dot(p2_ref[...], w2_ref[...],
                         preferred_element_type=jnp.float32) + b2_ref[...]


_RB = 1000
_MB = 640


def kernel(t, x, edge_index, W1, b1, W2, b2):
    del t
    f32 = jnp.float32

    edges = edge_index.astype(jnp.int32)
    W1p = jnp.zeros((D, HP), f32).at[:, :H].set(W1)
    W2p = jnp.zeros((HP, D), f32).at[:H, :].set(W2)
    b1v = jnp.zeros((16,), f32).at[:H].set(b1).at[8:8 + H].set(b1)
    b2r = b2.reshape(1, D)

    ones_c = jnp.ones((CHUNK, HP), f32)
    zeros_r = jnp.zeros((RPT, HP), f32)

    h1p, src_p, dst_p = pl.pallas_call(
        _mm1_body,
        grid=(NPAD // _MB,),
        in_specs=[pl.BlockSpec((_MB, D), lambda i: (i, 0)),
                  pl.BlockSpec((D, HP), lambda i: (0, 0)),
                  pl.BlockSpec((2, EC), lambda i: (0, i))],
        out_specs=[pl.BlockSpec((_MB, HP), lambda i: (i, 0)),
                   pl.BlockSpec((1, 1, EC), lambda i: (i, 0, 0)),
                   pl.BlockSpec((1, 1, EC), lambda i: (i, 0, 0))],
        out_shape=[jax.ShapeDtypeStruct((NPAD, HP), f32),
                   jax.ShapeDtypeStruct((NS, 1, EC), jnp.int32),
                   jax.ShapeDtypeStruct((NS, 1, EC), jnp.int32)],
    )(x, W1p, edges)
    src_t = src_p.reshape(NS, NCH, CHUNK)
    dst_t = dst_p.reshape(NS, NCH, CHUNK)

    p2 = _fused_sc(h1p, src_t, dst_t, ones_c, zeros_r, b1v)

    out = pl.pallas_call(
        _final_body,
        grid=(N // _RB,),
        in_specs=[pl.BlockSpec((_RB, HP), lambda i: (i, 0)),
                  pl.BlockSpec((HP, D), lambda i: (0, 0)),
                  pl.BlockSpec((1, D), lambda i: (0, 0))],
        out_specs=pl.BlockSpec((_RB, D), lambda i: (i, 0)),
        out_shape=jax.ShapeDtypeStruct((N, D), f32),
    )(p2, W2p, b2r)
    return out
